# baseline (speedup 1.0000x reference)
import functools

import jax
import jax.numpy as jnp
from jax import lax
from jax.experimental import pallas as pl
from jax.experimental.pallas import tpu as pltpu
from jax.experimental.pallas import tpu_sc as plsc

VOCAB = 1_000_000
D = 64
NSEQS = 16384
SEQ = 50
SEQ_PAD = 56
D_PAD = 128

NC, NS = 2, 16
NW = NC * NS

NSEQ = 8
NBUF = 2

SEQS_PER_W = NSEQS // NW
NSTEPS = SEQS_PER_W // NSEQ

_mesh = plsc.VectorSubcoreMesh(
    core_axis_name="c", subcore_axis_name="s", num_cores=NC, num_subcores=NS
)


def _pad_idx_body(i_ref, o_ref):
    x = i_ref[...]
    o_ref[...] = jnp.concatenate(
        [x, jnp.zeros((x.shape[0], D_PAD - SEQ), jnp.int32)], axis=1
    )


_pad_idx = pl.pallas_call(
    _pad_idx_body,
    out_shape=jax.ShapeDtypeStruct((NSEQS, D_PAD), jnp.int32),
    grid=(16,),
    in_specs=[pl.BlockSpec((NSEQS // 16, SEQ), lambda i: (i, 0))],
    out_specs=pl.BlockSpec((NSEQS // 16, D_PAD), lambda i: (i, 0)),
)


@functools.partial(
    pl.kernel,
    out_type=jax.ShapeDtypeStruct((NSEQS, SEQ_PAD, D_PAD), jnp.float32),
    mesh=_mesh,
    scratch_types=[
        pltpu.VMEM((NBUF, NSEQ, SEQ_PAD), jnp.int32),
        pltpu.VMEM((NBUF, NSEQ, SEQ_PAD, D), jnp.float32),
        pltpu.SemaphoreType.DMA,
        pltpu.SemaphoreType.DMA,
    ],
    compiler_params=pltpu.CompilerParams(use_tc_tiling_on_sc=False),
)
def _embed_gather(table_hbm, idx_hbm, out_hbm, idx_v, rows_v, sem0, sem1):
    sems = (sem0, sem1)
    wid = lax.axis_index("s") * NC + lax.axis_index("c")
    seq0 = wid * SEQS_PER_W

    def fire(slot, s):
        pltpu.sync_copy(
            idx_hbm.at[pl.ds(seq0 + s * NSEQ, NSEQ), pl.ds(0, SEQ_PAD)],
            idx_v.at[slot],
        )
        for j in range(NSEQ):
            pltpu.async_copy(
                table_hbm.at[idx_v.at[slot, j]],
                rows_v.at[slot, j],
                sems[slot],
            )

    def drain_flush(slot, s):
        pltpu.make_async_copy(
            out_hbm.at[pl.ds(0, NSEQ), pl.ds(0, SEQ_PAD), pl.ds(0, D)],
            rows_v.at[slot],
            sems[slot],
        ).wait()
        pltpu.sync_copy(
            rows_v.at[slot],
            out_hbm.at[
                pl.ds(seq0 + s * NSEQ, NSEQ), pl.ds(0, SEQ_PAD), pl.ds(0, D)
            ],
        )

    for b in range(NBUF):
        fire(b, b)

    @pl.loop(0, NSTEPS, step=NBUF)
    def _(g):
        for b in range(NBUF):
            s = g + b
            drain_flush(b, s)

            @pl.when(s + NBUF < NSTEPS)
            def _():
                fire(b, s + NBUF)


def kernel(token_ids, embeddings):
    idx_padded = _pad_idx(token_ids.astype(jnp.int32))
    out_padded = _embed_gather(embeddings, idx_padded)
    return out_padded[:, :SEQ, :D]
